# baseline (speedup 1.0000x reference)
import functools

import jax
import jax.numpy as jnp
from jax import lax
from jax.experimental import pallas as pl
from jax.experimental.pallas import tpu as pltpu
from jax.experimental.pallas import tpu_sc as plsc

B, L = 16384, 50
V, D = 100000, 128
H = 4096
NA, NS = 12, 3

LP = 64
NC, NSC = 2, 16
NW = NC * NSC
BPW = B // NW
CB = 2
NG = BPW // CB
ACHUNK = 128


def _sc_pool_body(x_hbm, aid_hbm, emb_hbm, aemb_hbm, avg_hbm, asp_hbm,
                  idx_v, rows_v, out_v, aidx_v, arows_v, sem):
    wid = lax.axis_index("s") * NC + lax.axis_index("c")
    base = wid * BPW

    def gather_step(g, carry):
        row0 = base + g * CB
        pltpu.sync_copy(x_hbm.at[pl.ds(row0 * LP, CB * LP)], idx_v)
        pltpu.async_copy(emb_hbm.at[idx_v], rows_v, sem).wait()
        for r in range(CB):
            U = 8
            def acc_step(j, accs):
                new = list(accs)
                for u in range(U):
                    row = r * LP + j * U + u
                    for c in range(D // 16):
                        new[c] = new[c] + rows_v[row, pl.ds(c * 16, 16)]
                return tuple(new)

            accs = lax.fori_loop(
                0, LP // U, acc_step,
                tuple(jnp.zeros((16,), jnp.float32) for _ in range(D // 16)))
            for c in range(D // 16):
                out_v[r, pl.ds(c * 16, 16)] = accs[c]
        pltpu.sync_copy(out_v, avg_hbm.at[pl.ds(row0, CB), :])
        return carry

    lax.fori_loop(0, NG, gather_step, 0)

    def aspect_step(q, carry):
        row0 = base + q * ACHUNK
        pltpu.sync_copy(aid_hbm.at[pl.ds(row0, ACHUNK)], aidx_v)
        pltpu.async_copy(aemb_hbm.at[aidx_v], arows_v, sem).wait()
        pltpu.sync_copy(arows_v, asp_hbm.at[pl.ds(row0, ACHUNK), :])
        return carry

    lax.fori_loop(0, BPW // ACHUNK, aspect_step, 0)


def _sc_pool(x_flat, aspect_ids, embedding, aspect_embedding):
    mesh = plsc.VectorSubcoreMesh(core_axis_name="c", subcore_axis_name="s")
    f = functools.partial(
        pl.kernel,
        mesh=mesh,
        out_type=[
            jax.ShapeDtypeStruct((B, D), jnp.float32),
            jax.ShapeDtypeStruct((B, D), jnp.float32),
        ],
        scratch_types=[
            pltpu.VMEM((CB * LP,), jnp.int32),
            pltpu.VMEM((CB * LP, D), jnp.float32),
            pltpu.VMEM((CB, D), jnp.float32),
            pltpu.VMEM((ACHUNK,), jnp.int32),
            pltpu.VMEM((ACHUNK, D), jnp.float32),
            pltpu.SemaphoreType.DMA,
        ],
    )(_sc_pool_body)
    return f(x_flat, aspect_ids, embedding, aspect_embedding)


def _mlp_body(sum_ref, asp_ref, x_ref, w1a_ref, w1b_ref, b1_ref, w2_ref,
              b2_ref, w3_ref, b3_ref, out_ref):
    cnt = jnp.sum((x_ref[...] != 0).astype(jnp.float32), axis=1, keepdims=True)
    avg = sum_ref[...] / jnp.maximum(cnt, 1.0)
    h1 = jnp.dot(avg, w1a_ref[...], preferred_element_type=jnp.float32)
    h1 = h1 + jnp.dot(asp_ref[...], w1b_ref[...],
                      preferred_element_type=jnp.float32)
    h1 = jnp.maximum(h1 + b1_ref[...], 0.0)
    h2 = jnp.dot(h1, w2_ref[...], preferred_element_type=jnp.float32)
    h2 = jnp.maximum(h2 + b2_ref[...], 0.0)
    out = jnp.dot(h2, w3_ref[...], preferred_element_type=jnp.float32)
    out_ref[...] = out + b3_ref[...]


def _mlp(emb_sum, asp, x2d, W1a, W1b, b1, W2, b2, W3, b3):
    BM = 512
    grid = (B // BM,)
    return pl.pallas_call(
        _mlp_body,
        grid=grid,
        in_specs=[
            pl.BlockSpec((BM, D), lambda i: (i, 0)),
            pl.BlockSpec((BM, D), lambda i: (i, 0)),
            pl.BlockSpec((BM, LP), lambda i: (i, 0)),
            pl.BlockSpec((D, H), lambda i: (0, 0)),
            pl.BlockSpec((D, H), lambda i: (0, 0)),
            pl.BlockSpec((1, H), lambda i: (0, 0)),
            pl.BlockSpec((H, H // 2), lambda i: (0, 0)),
            pl.BlockSpec((1, H // 2), lambda i: (0, 0)),
            pl.BlockSpec((H // 2, NS), lambda i: (0, 0)),
            pl.BlockSpec((1, NS), lambda i: (0, 0)),
        ],
        out_specs=pl.BlockSpec((BM, NS), lambda i: (i, 0)),
        out_shape=jax.ShapeDtypeStruct((B, NS), jnp.float32),
    )(emb_sum, asp, x2d, W1a, W1b, b1.reshape(1, H), W2,
      b2.reshape(1, H // 2), W3, b3.reshape(1, NS))


def kernel(x, aspect_ids, embedding, aspect_embedding, W1, b1, W2, b2, W3, b3):
    x2d = jnp.pad(x, ((0, 0), (0, LP - L)))
    x_flat = x2d.reshape(-1)
    emb_sum, asp = _sc_pool(x_flat, aspect_ids, embedding, aspect_embedding)
    return _mlp(emb_sum, asp, x2d, W1[:D], W1[D:], b1, W2, b2, W3, b3)
